# baseline (speedup 1.0000x reference)
import functools

import jax
import jax.numpy as jnp
from jax import lax
from jax.experimental import pallas as pl
from jax.experimental.pallas import tpu as pltpu
from jax.experimental.pallas import tpu_sc as plsc

N = 10000
E = 160000
R = 8
NB = 4
D = 256
LANES = 16
NCH = D // LANES
NC = 2
NS = 16
GRP = 128
GROUPS_PER_TILE = 80
TOT_GROUPS = NS * GROUPS_PER_TILE
E_PAD = TOT_GROUPS * GRP
NR = R * N
NR_PAD = NR + 256
ZROWS = NR_PAD // NS // 4
ROWS_PER_TILE = NR // NS
PAD_ROWS_PER_TILE = NR_PAD // NS
CHUNKS_PER_CORE = NCH // NC
NBUF = 2
SGLEN = 640
NSG = GROUPS_PER_TILE * GRP // SGLEN

_mesh = plsc.VectorSubcoreMesh(
    core_axis_name="c", subcore_axis_name="s", num_cores=NC, num_subcores=NS)
_sc_params = pltpu.CompilerParams(use_tc_tiling_on_sc=False)


def _agg_body(xcm, gidx, slot, zhbm, s_out, acc, gidx_v, slot_v, rows,
              gsem, ssem):
    core = lax.axis_index("c")
    tile = lax.axis_index("s")
    t0 = tile * NSG

    pltpu.sync_copy(slot.at[pl.ds(t0, NSG)], slot_v)

    def start_gather(sg, b):
        pltpu.async_copy(xcm.at[gidx_v.at[sg]], rows.at[b], gsem.at[b])

    def wait_gather(sg, b):
        pltpu.make_async_copy(xcm.at[gidx_v.at[sg]], rows.at[b],
                              gsem.at[b]).wait()

    def start_scatter(sg, b):
        pltpu.async_copy(rows.at[b], acc.at[slot_v.at[sg]], ssem.at[b],
                         add=True)

    def wait_scatter(sg, b):
        pltpu.make_async_copy(rows.at[b], acc.at[slot_v.at[sg]],
                              ssem.at[b]).wait()

    def chunk_body(cl, carry):
        c = core * CHUNKS_PER_CORE + cl
        pltpu.sync_copy(
            zhbm, acc.at[pl.ds(tile * PAD_ROWS_PER_TILE,
                               PAD_ROWS_PER_TILE)])
        plsc.subcore_barrier()
        pltpu.sync_copy(gidx.at[c, pl.ds(t0, NSG)], gidx_v)
        for b in range(NBUF):
            start_gather(b, b)
        for sg in range(NSG):
            b = sg % NBUF
            wait_gather(sg, b)
            start_scatter(sg, b)
            wait_scatter(sg, b)
            if sg + NBUF < NSG:
                start_gather(sg + NBUF, b)
        plsc.subcore_barrier()
        pltpu.sync_copy(
            acc.at[pl.ds(tile * ROWS_PER_TILE, ROWS_PER_TILE)],
            s_out.at[pl.ds(tile * ROWS_PER_TILE, ROWS_PER_TILE),
                     pl.ds(c * LANES, LANES)])
        plsc.subcore_barrier()
        return carry

    lax.fori_loop(0, CHUNKS_PER_CORE, chunk_body, None)


_agg_call = pl.kernel(
    _agg_body,
    out_type=jax.ShapeDtypeStruct((NR, D), jnp.float32),
    mesh=_mesh,
    scratch_types=[
        pltpu.VMEM_SHARED((NR_PAD, LANES), jnp.float32),
        pltpu.VMEM((NSG, SGLEN), jnp.int32),
        pltpu.VMEM((NSG, SGLEN), jnp.int32),
        pltpu.VMEM((NBUF, SGLEN, LANES), jnp.float32),
        pltpu.SemaphoreType.DMA((NBUF,)),
        pltpu.SemaphoreType.DMA((NBUF,)),
    ],
    compiler_params=_sc_params,
)


def _cnt_body(slot, zhbm, cnt_out, acc, slot_v, ones_v):
    core = lax.axis_index("c")
    tile = lax.axis_index("s")

    @pl.when(core == 0)
    def _():
        t0 = tile * NSG

        def oi(i, carry):
            ones_v[i, :] = jnp.ones((LANES,), jnp.float32)
            return carry
        lax.fori_loop(0, SGLEN, oi, None)

        pltpu.sync_copy(
            zhbm, acc.at[pl.ds(tile * PAD_ROWS_PER_TILE,
                               PAD_ROWS_PER_TILE)])
        pltpu.sync_copy(slot.at[pl.ds(t0, NSG)], slot_v)
        plsc.subcore_barrier()

        def grp_body(g, carry):
            pltpu.sync_copy(ones_v, acc.at[slot_v.at[g]], add=True)
            return carry
        lax.fori_loop(0, NSG, grp_body, None)
        plsc.subcore_barrier()
        pltpu.sync_copy(
            acc.at[pl.ds(tile * ROWS_PER_TILE, ROWS_PER_TILE)],
            cnt_out.at[pl.ds(tile * ROWS_PER_TILE, ROWS_PER_TILE)])


_cnt_call = pl.kernel(
    _cnt_body,
    out_type=jax.ShapeDtypeStruct((NR, LANES), jnp.float32),
    mesh=_mesh,
    scratch_types=[
        pltpu.VMEM_SHARED((NR_PAD, LANES), jnp.float32),
        pltpu.VMEM((NSG, SGLEN), jnp.int32),
        pltpu.VMEM((SGLEN, LANES), jnp.float32),
    ],
    compiler_params=_sc_params,
)

BN = 400


def _combine_body(act, emit_cm, s_ref, c_ref, x_ref, root_ref, basis_ref,
                  comp_ref, bias_ref, o_ref, *maybe_cm):
    xb = x_ref[...]
    acc = jnp.dot(xb, root_ref[...], preferred_element_type=jnp.float32)
    acc = acc + bias_ref[...]
    sns = []
    for r in range(R):
        inv = 1.0 / jnp.maximum(c_ref[r][:, 0:1], 1.0)
        sns.append(s_ref[r] * inv)
    for b in range(NB):
        t = sns[0] * comp_ref[0, b]
        for r in range(1, R):
            t = t + sns[r] * comp_ref[r, b]
        acc = acc + jnp.dot(t, basis_ref[b],
                            preferred_element_type=jnp.float32)
    if act == "relu":
        acc = jnp.maximum(acc, 0.0)
    elif act == "tanh":
        acc = jnp.tanh(acc)
    o_ref[...] = acc
    if emit_cm:
        cm_ref = maybe_cm[0]
        for c in range(NCH):
            cm_ref[c] = acc[:, c * LANES:(c + 1) * LANES]


def _combine(s3, c3, x, root, basis, comp, bias, act, h_out, emit_cm):
    out_specs = [pl.BlockSpec((BN, h_out), lambda i: (i, 0))]
    out_shape = [jax.ShapeDtypeStruct((N, h_out), jnp.float32)]
    if emit_cm:
        out_specs.append(pl.BlockSpec((NCH, BN, LANES), lambda i: (0, i, 0)))
        out_shape.append(
            jax.ShapeDtypeStruct((NCH, N, LANES), jnp.float32))
    return pl.pallas_call(
        functools.partial(_combine_body, act, emit_cm),
        grid=(N // BN,),
        in_specs=[
            pl.BlockSpec((R, BN, D), lambda i: (0, i, 0)),
            pl.BlockSpec((R, BN, LANES), lambda i: (0, i, 0)),
            pl.BlockSpec((BN, D), lambda i: (i, 0)),
            pl.BlockSpec((D, h_out), lambda i: (0, 0)),
            pl.BlockSpec((NB, D, h_out), lambda i: (0, 0, 0)),
            pl.BlockSpec(memory_space=pltpu.SMEM),
            pl.BlockSpec((1, h_out), lambda i: (0, 0)),
        ],
        out_specs=out_specs,
        out_shape=out_shape,
    )(s3, c3, x, root, basis, comp, bias.reshape(1, h_out))


def kernel(x, edge_index, edge_type, basis1, comp1, root1, bias1,
           basis2, comp2, root2, bias2, basis3, comp3, root3, bias3):
    src = edge_index[0].astype(jnp.int32)
    dst = edge_index[1].astype(jnp.int32)
    et = edge_type.astype(jnp.int32)
    order = jnp.argsort(et)
    src = src[order]
    dst = dst[order]
    et = et[order]
    slot = et * N + dst
    npad = E_PAD - E
    slot_p = jnp.concatenate(
        [slot, NR + (jnp.arange(npad, dtype=jnp.int32) % 256)])
    src_p = jnp.concatenate([src, jnp.zeros((npad,), jnp.int32)])
    slot2 = slot_p.reshape(NS * NSG, SGLEN)
    gidx = (src_p[None, :]
            + (jnp.arange(NCH, dtype=jnp.int32) * N)[:, None]).reshape(
                NCH, NS * NSG, SGLEN)

    zhbm = jnp.zeros((PAD_ROWS_PER_TILE, LANES), jnp.float32)
    counts = _cnt_call(slot2, zhbm)
    c3 = counts.reshape(R, N, LANES)

    h = x
    layers = [
        (basis1, comp1, root1, bias1, "relu", 256),
        (basis2, comp2, root2, bias2, "relu", 256),
        (basis3, comp3, root3, bias3, "tanh", 32),
    ]
    xcm = x.reshape(N, NCH, LANES).transpose(1, 0, 2).reshape(
        NCH * N, LANES)
    for li, (basis, comp, root, bias, act, h_out) in enumerate(layers):
        s = _agg_call(xcm, gidx, slot2, zhbm)
        res = _combine(s.reshape(R, N, D), c3, h, root, basis, comp, bias,
                       act, h_out, emit_cm=(li < 2))
        if li < 2:
            h, hcm = res
            xcm = hcm.reshape(NCH * N, LANES)
        else:
            h = res[0]
    return h


# speedup vs baseline: 1.6920x; 1.6920x over previous
import functools

import jax
import jax.numpy as jnp
from jax import lax
from jax.experimental import pallas as pl
from jax.experimental.pallas import tpu as pltpu
from jax.experimental.pallas import tpu_sc as plsc

N = 10000
E = 160000
R = 8
NB = 4
D = 256
LANES = 16
CW = 32
NCH = D // CW
NC = 2
NS = 16
GRP = 128
GROUPS_PER_TILE = 80
TOT_GROUPS = NS * GROUPS_PER_TILE
E_PAD = TOT_GROUPS * GRP
NR = R * N
NR_PAD = NR + 256
ZROWS = NR_PAD // NS // 4
ROWS_PER_TILE = NR // NS
PAD_ROWS_PER_TILE = NR_PAD // NS
CHUNKS_PER_CORE = NCH // NC
NBUF = 2
SGLEN = 640
NSG = GROUPS_PER_TILE * GRP // SGLEN

_mesh = plsc.VectorSubcoreMesh(
    core_axis_name="c", subcore_axis_name="s", num_cores=NC, num_subcores=NS)
_sc_params = pltpu.CompilerParams(use_tc_tiling_on_sc=False)


def _agg_body(xcm, gidx, slot, zbf, s_out, acc, gidx_v, slot_v, rows,
              gsem, ssem):
    core = lax.axis_index("c")
    tile = lax.axis_index("s")
    t0 = tile * NSG

    pltpu.sync_copy(slot.at[pl.ds(t0, NSG)], slot_v)

    def start_gather(sg, b):
        pltpu.async_copy(xcm.at[gidx_v.at[sg]], rows.at[b], gsem.at[b])

    def wait_gather(sg, b):
        pltpu.make_async_copy(xcm.at[gidx_v.at[sg]], rows.at[b],
                              gsem.at[b]).wait()

    def start_scatter(sg, b):
        pltpu.async_copy(rows.at[b], acc.at[slot_v.at[sg]], ssem.at[b],
                         add=True)

    def wait_scatter(sg, b):
        pltpu.make_async_copy(rows.at[b], acc.at[slot_v.at[sg]],
                              ssem.at[b]).wait()

    def chunk_body(cl, carry):
        c = core * CHUNKS_PER_CORE + cl
        pltpu.sync_copy(
            zbf, acc.at[pl.ds(tile * PAD_ROWS_PER_TILE,
                              PAD_ROWS_PER_TILE)])
        plsc.subcore_barrier()
        pltpu.sync_copy(gidx.at[c, pl.ds(t0, NSG)], gidx_v)
        for b in range(NBUF):
            start_gather(b, b)
        for sg in range(NSG):
            b = sg % NBUF
            wait_gather(sg, b)
            start_scatter(sg, b)
            wait_scatter(sg, b)
            if sg + NBUF < NSG:
                start_gather(sg + NBUF, b)
        plsc.subcore_barrier()
        pltpu.sync_copy(
            acc.at[pl.ds(tile * ROWS_PER_TILE, ROWS_PER_TILE)],
            s_out.at[pl.ds(tile * ROWS_PER_TILE, ROWS_PER_TILE),
                     pl.ds(c * CW, CW)])
        plsc.subcore_barrier()
        return carry

    lax.fori_loop(0, CHUNKS_PER_CORE, chunk_body, None)


_agg_call = pl.kernel(
    _agg_body,
    out_type=jax.ShapeDtypeStruct((NR, D), jnp.bfloat16),
    mesh=_mesh,
    scratch_types=[
        pltpu.VMEM_SHARED((NR_PAD, CW), jnp.bfloat16),
        pltpu.VMEM((NSG, SGLEN), jnp.int32),
        pltpu.VMEM((NSG, SGLEN), jnp.int32),
        pltpu.VMEM((NBUF, SGLEN, CW), jnp.bfloat16),
        pltpu.SemaphoreType.DMA((NBUF,)),
        pltpu.SemaphoreType.DMA((NBUF,)),
    ],
    compiler_params=_sc_params,
)


def _cnt_body(slot, zhbm, cnt_out, acc, slot_v, ones_v):
    core = lax.axis_index("c")
    tile = lax.axis_index("s")

    @pl.when(core == 0)
    def _():
        t0 = tile * NSG

        def oi(i, carry):
            ones_v[i, :] = jnp.ones((LANES,), jnp.float32)
            return carry
        lax.fori_loop(0, SGLEN, oi, None)

        pltpu.sync_copy(
            zhbm, acc.at[pl.ds(tile * PAD_ROWS_PER_TILE,
                               PAD_ROWS_PER_TILE)])
        pltpu.sync_copy(slot.at[pl.ds(t0, NSG)], slot_v)
        plsc.subcore_barrier()

        def grp_body(g, carry):
            pltpu.sync_copy(ones_v, acc.at[slot_v.at[g]], add=True)
            return carry
        lax.fori_loop(0, NSG, grp_body, None)
        plsc.subcore_barrier()
        pltpu.sync_copy(
            acc.at[pl.ds(tile * ROWS_PER_TILE, ROWS_PER_TILE)],
            cnt_out.at[pl.ds(tile * ROWS_PER_TILE, ROWS_PER_TILE)])


_cnt_call = pl.kernel(
    _cnt_body,
    out_type=jax.ShapeDtypeStruct((NR, LANES), jnp.float32),
    mesh=_mesh,
    scratch_types=[
        pltpu.VMEM_SHARED((NR_PAD, LANES), jnp.float32),
        pltpu.VMEM((NSG, SGLEN), jnp.int32),
        pltpu.VMEM((SGLEN, LANES), jnp.float32),
    ],
    compiler_params=_sc_params,
)

BN = 400


def _combine_body(act, emit_cm, s_ref, c_ref, x_ref, root_ref, basis_ref,
                  comp_ref, bias_ref, o_ref, *maybe_cm):
    xb = x_ref[...]
    acc = jnp.dot(xb, root_ref[...], preferred_element_type=jnp.float32)
    acc = acc + bias_ref[...]
    sns = []
    for r in range(R):
        inv = 1.0 / jnp.maximum(c_ref[r][:, 0:1], 1.0)
        sns.append(s_ref[r].astype(jnp.float32) * inv)
    for b in range(NB):
        t = sns[0] * comp_ref[0, b]
        for r in range(1, R):
            t = t + sns[r] * comp_ref[r, b]
        acc = acc + jnp.dot(t, basis_ref[b],
                            preferred_element_type=jnp.float32)
    if act == "relu":
        acc = jnp.maximum(acc, 0.0)
    elif act == "tanh":
        acc = jnp.tanh(acc)
    o_ref[...] = acc
    if emit_cm:
        cm_ref = maybe_cm[0]
        acc_bf = acc.astype(jnp.bfloat16)
        for c in range(NCH):
            cm_ref[c] = acc_bf[:, c * CW:(c + 1) * CW]


def _combine(s3, c3, x, root, basis, comp, bias, act, h_out, emit_cm):
    out_specs = [pl.BlockSpec((BN, h_out), lambda i: (i, 0))]
    out_shape = [jax.ShapeDtypeStruct((N, h_out), jnp.float32)]
    if emit_cm:
        out_specs.append(pl.BlockSpec((NCH, BN, CW), lambda i: (0, i, 0)))
        out_shape.append(
            jax.ShapeDtypeStruct((NCH, N, CW), jnp.bfloat16))
    return pl.pallas_call(
        functools.partial(_combine_body, act, emit_cm),
        grid=(N // BN,),
        in_specs=[
            pl.BlockSpec((R, BN, D), lambda i: (0, i, 0)),
            pl.BlockSpec((R, BN, LANES), lambda i: (0, i, 0)),
            pl.BlockSpec((BN, D), lambda i: (i, 0)),
            pl.BlockSpec((D, h_out), lambda i: (0, 0)),
            pl.BlockSpec((NB, D, h_out), lambda i: (0, 0, 0)),
            pl.BlockSpec(memory_space=pltpu.SMEM),
            pl.BlockSpec((1, h_out), lambda i: (0, 0)),
        ],
        out_specs=out_specs,
        out_shape=out_shape,
    )(s3, c3, x, root, basis, comp, bias.reshape(1, h_out))


def kernel(x, edge_index, edge_type, basis1, comp1, root1, bias1,
           basis2, comp2, root2, bias2, basis3, comp3, root3, bias3):
    src = edge_index[0].astype(jnp.int32)
    dst = edge_index[1].astype(jnp.int32)
    et = edge_type.astype(jnp.int32)
    slot = et * N + dst
    npad = E_PAD - E
    slot_p = jnp.concatenate(
        [slot, NR + (jnp.arange(npad, dtype=jnp.int32) % 256)])
    src_p = jnp.concatenate([src, jnp.zeros((npad,), jnp.int32)])
    slot2 = slot_p.reshape(NS * NSG, SGLEN)
    gidx = (src_p[None, :]
            + (jnp.arange(NCH, dtype=jnp.int32) * N)[:, None]).reshape(
                NCH, NS * NSG, SGLEN)

    zhbm = jnp.zeros((PAD_ROWS_PER_TILE, LANES), jnp.float32)
    zbf = jnp.zeros((PAD_ROWS_PER_TILE, CW), jnp.bfloat16)
    counts = _cnt_call(slot2, zhbm)
    c3 = counts.reshape(R, N, LANES)

    h = x
    layers = [
        (basis1, comp1, root1, bias1, "relu", 256),
        (basis2, comp2, root2, bias2, "relu", 256),
        (basis3, comp3, root3, bias3, "tanh", 32),
    ]
    xcm = x.astype(jnp.bfloat16).reshape(N, NCH, CW).transpose(
        1, 0, 2).reshape(NCH * N, CW)
    for li, (basis, comp, root, bias, act, h_out) in enumerate(layers):
        s = _agg_call(xcm, gidx, slot2, zbf)
        res = _combine(s.reshape(R, N, D), c3, h, root, basis, comp, bias,
                       act, h_out, emit_cm=(li < 2))
        if li < 2:
            h, hcm = res
            xcm = hcm.reshape(NCH * N, CW)
        else:
            h = res[0]
    return h


# speedup vs baseline: 1.9233x; 1.1367x over previous
import functools

import jax
import jax.numpy as jnp
from jax import lax
from jax.experimental import pallas as pl
from jax.experimental.pallas import tpu as pltpu
from jax.experimental.pallas import tpu_sc as plsc

N = 10000
E = 160000
R = 8
NB = 4
D = 256
LANES = 16
CW = 32
NCH = D // CW
NC = 2
NS = 16
GRP = 128
GROUPS_PER_TILE = 80
TOT_GROUPS = NS * GROUPS_PER_TILE
E_PAD = TOT_GROUPS * GRP
NR = R * N
NR_PAD = NR + 256
ZROWS = NR_PAD // NS // 4
ROWS_PER_TILE = NR // NS
PAD_ROWS_PER_TILE = NR_PAD // NS
CHUNKS_PER_CORE = NCH // NC
NBUF = 2
SGLEN = 512
NSG = GROUPS_PER_TILE * GRP // SGLEN
TROWS = N // NS

_mesh = plsc.VectorSubcoreMesh(
    core_axis_name="c", subcore_axis_name="s", num_cores=NC, num_subcores=NS)
_sc_params = pltpu.CompilerParams(use_tc_tiling_on_sc=False)


def _agg_body(xcm, gidx, slot, zbf, s_out, acc, tbl, gidx_v, slot_v, rows,
              gsem, ssem):
    core = lax.axis_index("c")
    tile = lax.axis_index("s")
    t0 = tile * NSG

    pltpu.sync_copy(slot.at[pl.ds(t0, NSG)], slot_v)
    pltpu.sync_copy(gidx.at[pl.ds(t0, NSG)], gidx_v)

    def start_gather(sg, b):
        pltpu.async_copy(tbl.at[gidx_v.at[sg]], rows.at[b], gsem.at[b])

    def wait_gather(sg, b):
        pltpu.make_async_copy(tbl.at[gidx_v.at[sg]], rows.at[b],
                              gsem.at[b]).wait()

    def start_scatter(sg, b):
        pltpu.async_copy(rows.at[b], acc.at[slot_v.at[sg]], ssem.at[b],
                         add=True)

    def wait_scatter(sg, b):
        pltpu.make_async_copy(rows.at[b], acc.at[slot_v.at[sg]],
                              ssem.at[b]).wait()

    def chunk_body(cl, carry):
        c = core * CHUNKS_PER_CORE + cl
        pltpu.sync_copy(
            zbf, acc.at[pl.ds(tile * PAD_ROWS_PER_TILE,
                              PAD_ROWS_PER_TILE)])
        pltpu.sync_copy(xcm.at[pl.ds(c * N + tile * TROWS, TROWS)],
                        tbl.at[pl.ds(tile * TROWS, TROWS)])
        plsc.subcore_barrier()
        for b in range(NBUF):
            start_gather(b, b)
        for sg in range(NSG):
            b = sg % NBUF
            wait_gather(sg, b)
            start_scatter(sg, b)
            wait_scatter(sg, b)
            if sg + NBUF < NSG:
                start_gather(sg + NBUF, b)
        plsc.subcore_barrier()
        pltpu.sync_copy(
            acc.at[pl.ds(tile * ROWS_PER_TILE, ROWS_PER_TILE)],
            s_out.at[pl.ds(tile * ROWS_PER_TILE, ROWS_PER_TILE),
                     pl.ds(c * CW, CW)])
        plsc.subcore_barrier()
        return carry

    lax.fori_loop(0, CHUNKS_PER_CORE, chunk_body, None)


_agg_call = pl.kernel(
    _agg_body,
    out_type=jax.ShapeDtypeStruct((NR, D), jnp.bfloat16),
    mesh=_mesh,
    scratch_types=[
        pltpu.VMEM_SHARED((NR_PAD, CW), jnp.bfloat16),
        pltpu.VMEM_SHARED((N, CW), jnp.bfloat16),
        pltpu.VMEM((NSG, SGLEN), jnp.int32),
        pltpu.VMEM((NSG, SGLEN), jnp.int32),
        pltpu.VMEM((NBUF, SGLEN, CW), jnp.bfloat16),
        pltpu.SemaphoreType.DMA((NBUF,)),
        pltpu.SemaphoreType.DMA((NBUF,)),
    ],
    compiler_params=_sc_params,
)


def _cnt_body(slot, zhbm, cnt_out, acc, slot_v, ones_v):
    core = lax.axis_index("c")
    tile = lax.axis_index("s")

    @pl.when(core == 0)
    def _():
        t0 = tile * NSG

        def oi(i, carry):
            ones_v[i, :] = jnp.ones((LANES,), jnp.float32)
            return carry
        lax.fori_loop(0, SGLEN, oi, None)

        pltpu.sync_copy(
            zhbm, acc.at[pl.ds(tile * PAD_ROWS_PER_TILE,
                               PAD_ROWS_PER_TILE)])
        pltpu.sync_copy(slot.at[pl.ds(t0, NSG)], slot_v)
        plsc.subcore_barrier()

        def grp_body(g, carry):
            pltpu.sync_copy(ones_v, acc.at[slot_v.at[g]], add=True)
            return carry
        lax.fori_loop(0, NSG, grp_body, None)
        plsc.subcore_barrier()
        pltpu.sync_copy(
            acc.at[pl.ds(tile * ROWS_PER_TILE, ROWS_PER_TILE)],
            cnt_out.at[pl.ds(tile * ROWS_PER_TILE, ROWS_PER_TILE)])


_cnt_call = pl.kernel(
    _cnt_body,
    out_type=jax.ShapeDtypeStruct((NR, LANES), jnp.float32),
    mesh=_mesh,
    scratch_types=[
        pltpu.VMEM_SHARED((NR_PAD, LANES), jnp.float32),
        pltpu.VMEM((NSG, SGLEN), jnp.int32),
        pltpu.VMEM((SGLEN, LANES), jnp.float32),
    ],
    compiler_params=_sc_params,
)

BN = 400


def _combine_body(act, emit_cm, s_ref, c_ref, x_ref, root_ref, basis_ref,
                  comp_ref, bias_ref, o_ref, *maybe_cm):
    xb = x_ref[...]
    acc = jnp.dot(xb, root_ref[...], preferred_element_type=jnp.float32)
    acc = acc + bias_ref[...]
    sns = []
    for r in range(R):
        inv = 1.0 / jnp.maximum(c_ref[r][:, 0:1], 1.0)
        sns.append(s_ref[r].astype(jnp.float32) * inv)
    for b in range(NB):
        t = sns[0] * comp_ref[0, b]
        for r in range(1, R):
            t = t + sns[r] * comp_ref[r, b]
        acc = acc + jnp.dot(t, basis_ref[b],
                            preferred_element_type=jnp.float32)
    if act == "relu":
        acc = jnp.maximum(acc, 0.0)
    elif act == "tanh":
        acc = jnp.tanh(acc)
    o_ref[...] = acc
    if emit_cm:
        cm_ref = maybe_cm[0]
        acc_bf = acc.astype(jnp.bfloat16)
        for c in range(NCH):
            cm_ref[c] = acc_bf[:, c * CW:(c + 1) * CW]


def _combine(s3, c3, x, root, basis, comp, bias, act, h_out, emit_cm):
    out_specs = [pl.BlockSpec((BN, h_out), lambda i: (i, 0))]
    out_shape = [jax.ShapeDtypeStruct((N, h_out), jnp.float32)]
    if emit_cm:
        out_specs.append(pl.BlockSpec((NCH, BN, CW), lambda i: (0, i, 0)))
        out_shape.append(
            jax.ShapeDtypeStruct((NCH, N, CW), jnp.bfloat16))
    return pl.pallas_call(
        functools.partial(_combine_body, act, emit_cm),
        grid=(N // BN,),
        in_specs=[
            pl.BlockSpec((R, BN, D), lambda i: (0, i, 0)),
            pl.BlockSpec((R, BN, LANES), lambda i: (0, i, 0)),
            pl.BlockSpec((BN, D), lambda i: (i, 0)),
            pl.BlockSpec((D, h_out), lambda i: (0, 0)),
            pl.BlockSpec((NB, D, h_out), lambda i: (0, 0, 0)),
            pl.BlockSpec(memory_space=pltpu.SMEM),
            pl.BlockSpec((1, h_out), lambda i: (0, 0)),
        ],
        out_specs=out_specs,
        out_shape=out_shape,
    )(s3, c3, x, root, basis, comp, bias.reshape(1, h_out))


def kernel(x, edge_index, edge_type, basis1, comp1, root1, bias1,
           basis2, comp2, root2, bias2, basis3, comp3, root3, bias3):
    src = edge_index[0].astype(jnp.int32)
    dst = edge_index[1].astype(jnp.int32)
    et = edge_type.astype(jnp.int32)
    slot = et * N + dst
    npad = E_PAD - E
    slot_p = jnp.concatenate(
        [slot, NR + (jnp.arange(npad, dtype=jnp.int32) % 256)])
    src_p = jnp.concatenate([src, jnp.zeros((npad,), jnp.int32)])
    slot2 = slot_p.reshape(NS * NSG, SGLEN)
    gidx = src_p.reshape(NS * NSG, SGLEN)

    zhbm = jnp.zeros((PAD_ROWS_PER_TILE, LANES), jnp.float32)
    zbf = jnp.zeros((PAD_ROWS_PER_TILE, CW), jnp.bfloat16)
    counts = _cnt_call(slot2, zhbm)
    c3 = counts.reshape(R, N, LANES)

    h = x
    layers = [
        (basis1, comp1, root1, bias1, "relu", 256),
        (basis2, comp2, root2, bias2, "relu", 256),
        (basis3, comp3, root3, bias3, "tanh", 32),
    ]
    xcm = x.astype(jnp.bfloat16).reshape(N, NCH, CW).transpose(
        1, 0, 2).reshape(NCH * N, CW)
    for li, (basis, comp, root, bias, act, h_out) in enumerate(layers):
        s = _agg_call(xcm, gidx, slot2, zbf)
        res = _combine(s.reshape(R, N, D), c3, h, root, basis, comp, bias,
                       act, h_out, emit_cm=(li < 2))
        if li < 2:
            h, hcm = res
            xcm = hcm.reshape(NCH * N, CW)
        else:
            h = res[0]
    return h
